# baseline (speedup 1.0000x reference)
import functools
import math

import jax
import jax.numpy as jnp
from jax import lax
from jax.experimental import pallas as pl
from jax.experimental.pallas import tpu as pltpu
import jax.experimental.pallas.tpu_sc as plsc

N = 10000
E = 320000
D = 128
NB = 128
L = 6
NG = 64
NATOM = 64
MAXR = 5.0
AVG_DEG = 32.0

NC = 2
NS = 16
NW = NC * NS
EPT = E // NW
C = 80
K = EPT // C
CA = 40
KA = EPT // CA
NP_ = 10240
RPT = NP_ // NS

_mesh = plsc.VectorSubcoreMesh(core_axis_name="c", subcore_axis_name="s")


@functools.partial(
    pl.kernel,
    out_type=jax.ShapeDtypeStruct((E,), jnp.float32),
    mesh=_mesh,
    compiler_params=pltpu.CompilerParams(needs_layout_passes=False),
    scratch_types=[
        pltpu.VMEM((C,), jnp.int32),
        pltpu.VMEM((C,), jnp.int32),
        pltpu.VMEM((6, C), jnp.float32),
        pltpu.VMEM((EPT,), jnp.float32),
        pltpu.SemaphoreType.DMA,
    ],
)
def _sc_d2(px_hbm, py_hbm, pz_hbm, src_hbm, dst_hbm, out_hbm,
           src_v, dst_v, g_v, d2_v, sem):
    c = lax.axis_index("c")
    s = lax.axis_index("s")
    w = c * NS + s
    base = w * EPT

    def chunk(k, _):
        pltpu.sync_copy(src_hbm.at[pl.ds(base + k * C, C)], src_v)
        pltpu.sync_copy(dst_hbm.at[pl.ds(base + k * C, C)], dst_v)
        cps = [
            pltpu.async_copy(px_hbm.at[src_v], g_v.at[0], sem),
            pltpu.async_copy(py_hbm.at[src_v], g_v.at[1], sem),
            pltpu.async_copy(pz_hbm.at[src_v], g_v.at[2], sem),
            pltpu.async_copy(px_hbm.at[dst_v], g_v.at[3], sem),
            pltpu.async_copy(py_hbm.at[dst_v], g_v.at[4], sem),
            pltpu.async_copy(pz_hbm.at[dst_v], g_v.at[5], sem),
        ]
        for cp in cps:
            cp.wait()
        for t in range(C // 16):
            sl = pl.ds(t * 16, 16)
            dx = g_v[0, sl] - g_v[3, sl]
            dy = g_v[1, sl] - g_v[4, sl]
            dz = g_v[2, sl] - g_v[5, sl]
            d2_v[pl.ds(k * C + t * 16, 16)] = dx * dx + dy * dy + dz * dz
        return 0

    lax.fori_loop(0, K, chunk, 0)
    pltpu.sync_copy(d2_v, out_hbm.at[pl.ds(base, EPT)])


_M16 = -65536


@functools.partial(
    pl.kernel,
    out_type=jax.ShapeDtypeStruct((NC, NP_, D), jnp.float32),
    mesh=_mesh,
    compiler_params=pltpu.CompilerParams(
        needs_layout_passes=False, use_tc_tiling_on_sc=False),
    scratch_types=[
        pltpu.VMEM_SHARED((NP_, D), jnp.float32),
        pltpu.VMEM((EPT,), jnp.int32),
        pltpu.VMEM((EPT,), jnp.int32),
        pltpu.VMEM((CA, D // 2), jnp.int32),
        pltpu.VMEM((CA, D // 2), jnp.int32),
        pltpu.VMEM((CA, D // 2), jnp.int32),
        pltpu.VMEM((CA, D // 2), jnp.int32),
        pltpu.VMEM((CA, D), jnp.float32),
        pltpu.VMEM((CA, D), jnp.float32),
        pltpu.SemaphoreType.DMA,
        pltpu.SemaphoreType.DMA,
        pltpu.SemaphoreType.DMA,
        pltpu.SemaphoreType.DMA,
        pltpu.SemaphoreType.DMA,
        pltpu.SemaphoreType.DMA,
    ],
)
def _sc_agg(h_hbm, ew_hbm, src_hbm, dst_hbm, zeros_hbm, out_hbm,
            agg_sh, srcall, dstall, msg0, msg1, ew0, ew1, sb0, sb1,
            sg0, sg1, se0, se1, ss0, ss1):
    c = lax.axis_index("c")
    s = lax.axis_index("s")
    w = c * NS + s
    base = w * EPT
    row0 = s * RPT
    msgb = (msg0, msg1)
    ewb = (ew0, ew1)
    sbb = (sb0, sb1)
    sgb = (sg0, sg1)
    seb = (se0, se1)
    ssb = (ss0, ss1)

    pltpu.sync_copy(zeros_hbm, agg_sh.at[pl.ds(row0, RPT)])
    pltpu.sync_copy(src_hbm.at[pl.ds(base, EPT)], srcall)
    pltpu.sync_copy(dst_hbm.at[pl.ds(base, EPT)], dstall)
    plsc.subcore_barrier()

    def issue(k, b):
        pltpu.async_copy(h_hbm.at[srcall.at[pl.ds(k * CA, CA)]], msgb[b], sgb[b])
        pltpu.async_copy(ew_hbm.at[pl.ds(base + k * CA, CA)], ewb[b], seb[b])

    def wait_scatter(b):
        pltpu.make_async_copy(
            sbb[b], agg_sh.at[dstall.at[pl.ds(0, CA)]], ssb[b]).wait()

    def process(k, b):
        pltpu.make_async_copy(
            h_hbm.at[srcall.at[pl.ds(0, CA)]], msgb[b], sgb[b]).wait()
        pltpu.make_async_copy(
            ew_hbm.at[pl.ds(0, CA)], ewb[b], seb[b]).wait()

        def rowbody(r, _):
            for j in range(D // 32):
                vh = msgb[b][r, pl.ds(j * 16, 16)]
                ve = ewb[b][pl.ds(r * (D // 2) + j * 16, 16)]
                he = plsc.bitcast(vh << 16, jnp.float32)
                ho = plsc.bitcast(vh & _M16, jnp.float32)
                ee = plsc.bitcast(ve << 16, jnp.float32)
                eo = plsc.bitcast(ve & _M16, jnp.float32)
                sbb[b][r, pl.ds(j * 32, 16)] = he * ee
                sbb[b][r, pl.ds(j * 32 + 16, 16)] = ho * eo
            return 0

        def rowbody(r2, _):
            for rr in range(2):
                r = r2 * 2 + rr
                for j in range(D // 32):
                    vh = msgb[b][r, pl.ds(j * 16, 16)]
                    ve = ewb[b][r, pl.ds(j * 16, 16)]
                    he = plsc.bitcast(vh << 16, jnp.float32)
                    ho = plsc.bitcast(vh & _M16, jnp.float32)
                    ee = plsc.bitcast(ve << 16, jnp.float32)
                    eo = plsc.bitcast(ve & _M16, jnp.float32)
                    sbb[b][r, pl.ds(j * 32, 16)] = he * ee
                    sbb[b][r, pl.ds(j * 32 + 16, 16)] = ho * eo
            return 0

        lax.fori_loop(0, CA // 2, rowbody, 0)
        pltpu.async_copy(sbb[b], agg_sh.at[dstall.at[pl.ds(k * CA, CA)]],
                         ssb[b], add=True)

    issue(0, 0)
    issue(1, 1)

    def pair(i, _):
        for b in range(2):
            @pl.when(i > 0)
            def _():
                wait_scatter(b)

            process(2 * i + b, b)

            @pl.when(i + 1 < KA // 2)
            def _():
                issue(2 * i + 2 + b, b)
        return 0

    lax.fori_loop(0, KA // 2, pair, 0)
    wait_scatter(0)
    wait_scatter(1)
    plsc.subcore_barrier()
    pltpu.sync_copy(agg_sh.at[pl.ds(row0, RPT)],
                    out_hbm.at[c, pl.ds(row0, RPT)])


_RB = 2000
_GN = N // _RB
_EB = 2000
_GE = E // _EB


def _pack16(a, b):
    ai = lax.bitcast_convert_type(
        a.astype(jnp.bfloat16).astype(jnp.float32), jnp.int32)
    bi = lax.bitcast_convert_type(
        b.astype(jnp.bfloat16).astype(jnp.float32), jnp.int32)
    return lax.shift_right_logical(ai, 16) | (bi & _M16)


def _tc_ew_body(d2_ref, cen_ref, w1_ref, w2_ref, se_ref, so_ref, out_ref):
    d2 = d2_ref[...]
    dist = jnp.sqrt(d2 + 1e-8)
    cen = cen_ref[...]
    width = MAXR / NB
    rbf = jnp.exp(-((dist - cen) ** 2) * (1.0 / (2.0 * width * width)))
    h1 = jax.nn.silu(jnp.dot(rbf, w1_ref[...], preferred_element_type=jnp.float32))
    ew = jnp.dot(h1, w2_ref[...], preferred_element_type=jnp.float32)
    out_ref[...] = _pack16(
        jnp.dot(ew, se_ref[...], preferred_element_type=jnp.float32),
        jnp.dot(ew, so_ref[...], preferred_element_type=jnp.float32))


def _tc_ew(d2, centers, w1, w2, sel_e, sel_o):
    return pl.pallas_call(
        _tc_ew_body,
        grid=(_GE,),
        in_specs=[
            pl.BlockSpec((_EB, 1), lambda i: (i, 0)),
            pl.BlockSpec((1, NB), lambda i: (0, 0)),
            pl.BlockSpec((NB, 64), lambda i: (0, 0)),
            pl.BlockSpec((64, D), lambda i: (0, 0)),
            pl.BlockSpec((D, D // 2), lambda i: (0, 0)),
            pl.BlockSpec((D, D // 2), lambda i: (0, 0)),
        ],
        out_specs=pl.BlockSpec((_EB, D // 2), lambda i: (i, 0)),
        out_shape=jax.ShapeDtypeStruct((E, D // 2), jnp.int32),
    )(d2, centers, w1, w2, sel_e, sel_o)


def _layernorm(x, w, b):
    mu = jnp.mean(x, axis=-1, keepdims=True)
    var = jnp.var(x, axis=-1, keepdims=True)
    return (x - mu) / jnp.sqrt(var + 1e-5) * w + b


def _tc_embed_body(na_ref, tab_ref, lnw_ref, lnb_ref, se_ref, so_ref,
                   x_ref, h_ref):
    na = na_ref[...]
    ids = lax.broadcasted_iota(jnp.int32, (_RB, NATOM), 1)
    onehot = (na == ids).astype(jnp.float32)
    x = jnp.dot(onehot, tab_ref[...], preferred_element_type=jnp.float32)
    x_ref[...] = x
    h = _layernorm(x, lnw_ref[...], lnb_ref[...])
    h_ref[...] = _pack16(
        jnp.dot(h, se_ref[...], preferred_element_type=jnp.float32),
        jnp.dot(h, so_ref[...], preferred_element_type=jnp.float32))


def _tc_embed(na, tab, lnw0, lnb0, sel_e, sel_o):
    return pl.pallas_call(
        _tc_embed_body,
        grid=(_GN,),
        in_specs=[
            pl.BlockSpec((_RB, 1), lambda i: (i, 0)),
            pl.BlockSpec((NATOM, D), lambda i: (0, 0)),
            pl.BlockSpec((1, D), lambda i: (0, 0)),
            pl.BlockSpec((1, D), lambda i: (0, 0)),
            pl.BlockSpec((D, D // 2), lambda i: (0, 0)),
            pl.BlockSpec((D, D // 2), lambda i: (0, 0)),
        ],
        out_specs=[
            pl.BlockSpec((_RB, D), lambda i: (i, 0)),
            pl.BlockSpec((_RB, D // 2), lambda i: (i, 0)),
        ],
        out_shape=[
            jax.ShapeDtypeStruct((N, D), jnp.float32),
            jax.ShapeDtypeStruct((N, D // 2), jnp.int32),
        ],
    )(na, tab, lnw0, lnb0, sel_e, sel_o)


def _tc_update_body(x_ref, agg_ref, w_ref, lnw_ref, lnb_ref, se_ref, so_ref,
                    xn_ref, hn_ref):
    agg = (agg_ref[0] + agg_ref[1]) * (1.0 / math.sqrt(AVG_DEG))
    up = jnp.dot(agg, w_ref[...], preferred_element_type=jnp.float32)
    xn = x_ref[...] + jax.nn.silu(up)
    xn_ref[...] = xn
    hn = _layernorm(xn, lnw_ref[...], lnb_ref[...])
    hn_ref[...] = _pack16(
        jnp.dot(hn, se_ref[...], preferred_element_type=jnp.float32),
        jnp.dot(hn, so_ref[...], preferred_element_type=jnp.float32))


def _tc_update(x, aggp, w, lnw, lnb, sel_e, sel_o):
    return pl.pallas_call(
        _tc_update_body,
        grid=(_GN,),
        in_specs=[
            pl.BlockSpec((_RB, D), lambda i: (i, 0)),
            pl.BlockSpec((NC, _RB, D), lambda i: (0, i, 0)),
            pl.BlockSpec((D, D), lambda i: (0, 0)),
            pl.BlockSpec((1, D), lambda i: (0, 0)),
            pl.BlockSpec((1, D), lambda i: (0, 0)),
            pl.BlockSpec((D, D // 2), lambda i: (0, 0)),
            pl.BlockSpec((D, D // 2), lambda i: (0, 0)),
        ],
        out_specs=[
            pl.BlockSpec((_RB, D), lambda i: (i, 0)),
            pl.BlockSpec((_RB, D // 2), lambda i: (i, 0)),
        ],
        out_shape=[
            jax.ShapeDtypeStruct((N, D), jnp.float32),
            jax.ShapeDtypeStruct((N, D // 2), jnp.int32),
        ],
    )(x, aggp, w, lnw, lnb, sel_e, sel_o)


def _tc_head_body(x_ref, wh_ref, b_ref, out_ref):
    i = pl.program_id(0)

    @pl.when(i == 0)
    def _():
        out_ref[...] = jnp.zeros_like(out_ref)

    e_node = jnp.dot(x_ref[...], wh_ref[...], preferred_element_type=jnp.float32)
    ids = lax.broadcasted_iota(jnp.int32, (_RB, NG), 1)
    onehot = (b_ref[...] == ids).astype(jnp.float32)
    out_ref[...] += lax.dot_general(
        onehot, e_node, (((0,), (0,)), ((), ())),
        preferred_element_type=jnp.float32)


def _tc_head(x, wh, batch2):
    return pl.pallas_call(
        _tc_head_body,
        grid=(_GN,),
        in_specs=[
            pl.BlockSpec((_RB, D), lambda i: (i, 0)),
            pl.BlockSpec((D, 1), lambda i: (0, 0)),
            pl.BlockSpec((_RB, 1), lambda i: (i, 0)),
        ],
        out_specs=pl.BlockSpec((NG, 1), lambda i: (0, 0)),
        out_shape=jax.ShapeDtypeStruct((NG, 1), jnp.float32),
    )(x, wh, batch2)


_QPERM = []
for _g in range(D // 32):
    _QPERM += list(range(32 * _g, 32 * _g + 32, 2))
    _QPERM += list(range(32 * _g + 1, 32 * _g + 32, 2))


def kernel(pos, node_atom, batch, edge_index, atom_table, rbf_centers,
           W_rbf1, W_rbf2, W_layers, ln_w, ln_b, W_head):
    src = edge_index[0].astype(jnp.int32)
    dst = edge_index[1].astype(jnp.int32)
    px = pos[:, 0]
    py = pos[:, 1]
    pz = pos[:, 2]
    zeros = jnp.zeros((RPT, D), jnp.float32)
    na2 = node_atom.astype(jnp.int32).reshape(N, 1)
    b2 = batch.astype(jnp.int32).reshape(N, 1)

    ar = jnp.arange(D // 2)
    sel_e = jnp.zeros((D, D // 2), jnp.float32).at[2 * ar, ar].set(1.0)
    sel_o = jnp.zeros((D, D // 2), jnp.float32).at[2 * ar + 1, ar].set(1.0)
    w_q = W_layers[:, jnp.array(_QPERM), :]

    d2 = _sc_d2(px, py, pz, src, dst)
    ewp = _tc_ew(d2.reshape(E, 1), rbf_centers.reshape(1, NB),
                 W_rbf1, W_rbf2, sel_e, sel_o)

    x, h = _tc_embed(na2, atom_table, ln_w[0].reshape(1, D),
                     ln_b[0].reshape(1, D), sel_e, sel_o)
    for l in range(L):
        aggp = _sc_agg(h, ewp, src, dst, zeros)
        nl = (l + 1) % L
        x, h = _tc_update(x, aggp, w_q[l], ln_w[nl].reshape(1, D),
                          ln_b[nl].reshape(1, D), sel_e, sel_o)
    return _tc_head(x, W_head, b2)


# speedup vs baseline: 1.4385x; 1.4385x over previous
import functools
import math

import jax
import jax.numpy as jnp
from jax import lax
from jax.experimental import pallas as pl
from jax.experimental.pallas import tpu as pltpu
import jax.experimental.pallas.tpu_sc as plsc

N = 10000
E = 320000
D = 128
NB = 128
L = 6
NG = 64
NATOM = 64
MAXR = 5.0
AVG_DEG = 32.0

NC = 2
NS = 16
NW = NC * NS
EPT = E // NW
C = 80
K = EPT // C
CA = 40
KA = EPT // CA
NP_ = 10240
RPT = NP_ // NS

_mesh = plsc.VectorSubcoreMesh(core_axis_name="c", subcore_axis_name="s")


@functools.partial(
    pl.kernel,
    out_type=jax.ShapeDtypeStruct((E,), jnp.float32),
    mesh=_mesh,
    compiler_params=pltpu.CompilerParams(needs_layout_passes=False),
    scratch_types=[
        pltpu.VMEM((C,), jnp.int32),
        pltpu.VMEM((C,), jnp.int32),
        pltpu.VMEM((6, C), jnp.float32),
        pltpu.VMEM((EPT,), jnp.float32),
        pltpu.SemaphoreType.DMA,
    ],
)
def _sc_d2(px_hbm, py_hbm, pz_hbm, src_hbm, dst_hbm, out_hbm,
           src_v, dst_v, g_v, d2_v, sem):
    c = lax.axis_index("c")
    s = lax.axis_index("s")
    w = c * NS + s
    base = w * EPT

    def chunk(k, _):
        pltpu.sync_copy(src_hbm.at[pl.ds(base + k * C, C)], src_v)
        pltpu.sync_copy(dst_hbm.at[pl.ds(base + k * C, C)], dst_v)
        cps = [
            pltpu.async_copy(px_hbm.at[src_v], g_v.at[0], sem),
            pltpu.async_copy(py_hbm.at[src_v], g_v.at[1], sem),
            pltpu.async_copy(pz_hbm.at[src_v], g_v.at[2], sem),
            pltpu.async_copy(px_hbm.at[dst_v], g_v.at[3], sem),
            pltpu.async_copy(py_hbm.at[dst_v], g_v.at[4], sem),
            pltpu.async_copy(pz_hbm.at[dst_v], g_v.at[5], sem),
        ]
        for cp in cps:
            cp.wait()
        for t in range(C // 16):
            sl = pl.ds(t * 16, 16)
            dx = g_v[0, sl] - g_v[3, sl]
            dy = g_v[1, sl] - g_v[4, sl]
            dz = g_v[2, sl] - g_v[5, sl]
            d2_v[pl.ds(k * C + t * 16, 16)] = dx * dx + dy * dy + dz * dz
        return 0

    lax.fori_loop(0, K, chunk, 0)
    pltpu.sync_copy(d2_v, out_hbm.at[pl.ds(base, EPT)])


_M16 = -65536


@functools.partial(
    pl.kernel,
    out_type=jax.ShapeDtypeStruct((NC, NP_, D), jnp.float32),
    mesh=_mesh,
    compiler_params=pltpu.CompilerParams(
        needs_layout_passes=False, use_tc_tiling_on_sc=False),
    scratch_types=[
        pltpu.VMEM_SHARED((NP_, D), jnp.float32),
        pltpu.VMEM((EPT,), jnp.int32),
        pltpu.VMEM((EPT,), jnp.int32),
        pltpu.VMEM((CA, D // 2), jnp.int32),
        pltpu.VMEM((CA, D // 2), jnp.int32),
        pltpu.VMEM((CA, D // 2), jnp.int32),
        pltpu.VMEM((CA, D // 2), jnp.int32),
        pltpu.VMEM((CA, D), jnp.float32),
        pltpu.VMEM((CA, D), jnp.float32),
        pltpu.SemaphoreType.DMA,
        pltpu.SemaphoreType.DMA,
        pltpu.SemaphoreType.DMA,
        pltpu.SemaphoreType.DMA,
        pltpu.SemaphoreType.DMA,
        pltpu.SemaphoreType.DMA,
    ],
)
def _sc_agg(h_hbm, ew_hbm, src_hbm, dst_hbm, zeros_hbm, out_hbm,
            agg_sh, srcall, dstall, msg0, msg1, ew0, ew1, sb0, sb1,
            sg0, sg1, se0, se1, ss0, ss1):
    c = lax.axis_index("c")
    s = lax.axis_index("s")
    w = c * NS + s
    base = w * EPT
    row0 = s * RPT
    msgb = (msg0, msg1)
    ewb = (ew0, ew1)
    sbb = (sb0, sb1)
    sgb = (sg0, sg1)
    seb = (se0, se1)
    ssb = (ss0, ss1)

    pltpu.sync_copy(zeros_hbm, agg_sh.at[pl.ds(row0, RPT)])
    pltpu.sync_copy(src_hbm.at[pl.ds(base, EPT)], srcall)
    pltpu.sync_copy(dst_hbm.at[pl.ds(base, EPT)], dstall)
    plsc.subcore_barrier()

    def issue(k, b):
        pltpu.async_copy(h_hbm.at[srcall.at[pl.ds(k * CA, CA)]], msgb[b], sgb[b])
        pltpu.async_copy(ew_hbm.at[pl.ds(base + k * CA, CA)], ewb[b], seb[b])

    def wait_scatter(b):
        pltpu.make_async_copy(
            sbb[b], agg_sh.at[dstall.at[pl.ds(0, CA)]], ssb[b]).wait()

    def process(k, b):
        pltpu.make_async_copy(
            h_hbm.at[srcall.at[pl.ds(0, CA)]], msgb[b], sgb[b]).wait()
        pltpu.make_async_copy(
            ew_hbm.at[pl.ds(0, CA)], ewb[b], seb[b]).wait()

        def rowbody(r, _):
            for j in range(D // 32):
                vh = msgb[b][r, pl.ds(j * 16, 16)]
                ve = ewb[b][pl.ds(r * (D // 2) + j * 16, 16)]
                he = plsc.bitcast(vh << 16, jnp.float32)
                ho = plsc.bitcast(vh & _M16, jnp.float32)
                ee = plsc.bitcast(ve << 16, jnp.float32)
                eo = plsc.bitcast(ve & _M16, jnp.float32)
                sbb[b][r, pl.ds(j * 32, 16)] = he * ee
                sbb[b][r, pl.ds(j * 32 + 16, 16)] = ho * eo
            return 0

        @plsc.parallel_loop(0, CA, 1, unroll=2)
        def rowbody(r):
            for j in range(D // 32):
                vh = msgb[b][r, pl.ds(j * 16, 16)]
                ve = ewb[b][r, pl.ds(j * 16, 16)]
                he = plsc.bitcast(vh << 16, jnp.float32)
                ho = plsc.bitcast(vh & _M16, jnp.float32)
                ee = plsc.bitcast(ve << 16, jnp.float32)
                eo = plsc.bitcast(ve & _M16, jnp.float32)
                sbb[b][r, pl.ds(j * 32, 16)] = he * ee
                sbb[b][r, pl.ds(j * 32 + 16, 16)] = ho * eo
        pltpu.async_copy(sbb[b], agg_sh.at[dstall.at[pl.ds(k * CA, CA)]],
                         ssb[b], add=True)

    issue(0, 0)
    issue(1, 1)

    def pair(i, _):
        for b in range(2):
            @pl.when(i > 0)
            def _():
                wait_scatter(b)

            process(2 * i + b, b)

            @pl.when(i + 1 < KA // 2)
            def _():
                issue(2 * i + 2 + b, b)
        return 0

    lax.fori_loop(0, KA // 2, pair, 0)
    wait_scatter(0)
    wait_scatter(1)
    plsc.subcore_barrier()
    pltpu.sync_copy(agg_sh.at[pl.ds(row0, RPT)],
                    out_hbm.at[c, pl.ds(row0, RPT)])


_RB = 2000
_GN = N // _RB
_EB = 2000
_GE = E // _EB


def _pack16(a, b):
    ai = lax.bitcast_convert_type(
        a.astype(jnp.bfloat16).astype(jnp.float32), jnp.int32)
    bi = lax.bitcast_convert_type(
        b.astype(jnp.bfloat16).astype(jnp.float32), jnp.int32)
    return lax.shift_right_logical(ai, 16) | (bi & _M16)


def _tc_ew_body(d2_ref, cen_ref, w1_ref, w2_ref, se_ref, so_ref, out_ref):
    d2 = d2_ref[...]
    dist = jnp.sqrt(d2 + 1e-8)
    cen = cen_ref[...]
    width = MAXR / NB
    rbf = jnp.exp(-((dist - cen) ** 2) * (1.0 / (2.0 * width * width)))
    h1 = jax.nn.silu(jnp.dot(rbf, w1_ref[...], preferred_element_type=jnp.float32))
    ew = jnp.dot(h1, w2_ref[...], preferred_element_type=jnp.float32)
    out_ref[...] = _pack16(
        jnp.dot(ew, se_ref[...], preferred_element_type=jnp.float32),
        jnp.dot(ew, so_ref[...], preferred_element_type=jnp.float32))


def _tc_ew(d2, centers, w1, w2, sel_e, sel_o):
    return pl.pallas_call(
        _tc_ew_body,
        grid=(_GE,),
        in_specs=[
            pl.BlockSpec((_EB, 1), lambda i: (i, 0)),
            pl.BlockSpec((1, NB), lambda i: (0, 0)),
            pl.BlockSpec((NB, 64), lambda i: (0, 0)),
            pl.BlockSpec((64, D), lambda i: (0, 0)),
            pl.BlockSpec((D, D // 2), lambda i: (0, 0)),
            pl.BlockSpec((D, D // 2), lambda i: (0, 0)),
        ],
        out_specs=pl.BlockSpec((_EB, D // 2), lambda i: (i, 0)),
        out_shape=jax.ShapeDtypeStruct((E, D // 2), jnp.int32),
    )(d2, centers, w1, w2, sel_e, sel_o)


def _layernorm(x, w, b):
    mu = jnp.mean(x, axis=-1, keepdims=True)
    var = jnp.var(x, axis=-1, keepdims=True)
    return (x - mu) / jnp.sqrt(var + 1e-5) * w + b


def _tc_embed_body(na_ref, tab_ref, lnw_ref, lnb_ref, se_ref, so_ref,
                   x_ref, h_ref):
    na = na_ref[...]
    ids = lax.broadcasted_iota(jnp.int32, (_RB, NATOM), 1)
    onehot = (na == ids).astype(jnp.float32)
    x = jnp.dot(onehot, tab_ref[...], preferred_element_type=jnp.float32)
    x_ref[...] = x
    h = _layernorm(x, lnw_ref[...], lnb_ref[...])
    h_ref[...] = _pack16(
        jnp.dot(h, se_ref[...], preferred_element_type=jnp.float32),
        jnp.dot(h, so_ref[...], preferred_element_type=jnp.float32))


def _tc_embed(na, tab, lnw0, lnb0, sel_e, sel_o):
    return pl.pallas_call(
        _tc_embed_body,
        grid=(_GN,),
        in_specs=[
            pl.BlockSpec((_RB, 1), lambda i: (i, 0)),
            pl.BlockSpec((NATOM, D), lambda i: (0, 0)),
            pl.BlockSpec((1, D), lambda i: (0, 0)),
            pl.BlockSpec((1, D), lambda i: (0, 0)),
            pl.BlockSpec((D, D // 2), lambda i: (0, 0)),
            pl.BlockSpec((D, D // 2), lambda i: (0, 0)),
        ],
        out_specs=[
            pl.BlockSpec((_RB, D), lambda i: (i, 0)),
            pl.BlockSpec((_RB, D // 2), lambda i: (i, 0)),
        ],
        out_shape=[
            jax.ShapeDtypeStruct((N, D), jnp.float32),
            jax.ShapeDtypeStruct((N, D // 2), jnp.int32),
        ],
    )(na, tab, lnw0, lnb0, sel_e, sel_o)


def _tc_update_body(x_ref, agg_ref, w_ref, lnw_ref, lnb_ref, se_ref, so_ref,
                    xn_ref, hn_ref):
    agg = (agg_ref[0] + agg_ref[1]) * (1.0 / math.sqrt(AVG_DEG))
    up = jnp.dot(agg, w_ref[...], preferred_element_type=jnp.float32)
    xn = x_ref[...] + jax.nn.silu(up)
    xn_ref[...] = xn
    hn = _layernorm(xn, lnw_ref[...], lnb_ref[...])
    hn_ref[...] = _pack16(
        jnp.dot(hn, se_ref[...], preferred_element_type=jnp.float32),
        jnp.dot(hn, so_ref[...], preferred_element_type=jnp.float32))


def _tc_update(x, aggp, w, lnw, lnb, sel_e, sel_o):
    return pl.pallas_call(
        _tc_update_body,
        grid=(_GN,),
        in_specs=[
            pl.BlockSpec((_RB, D), lambda i: (i, 0)),
            pl.BlockSpec((NC, _RB, D), lambda i: (0, i, 0)),
            pl.BlockSpec((D, D), lambda i: (0, 0)),
            pl.BlockSpec((1, D), lambda i: (0, 0)),
            pl.BlockSpec((1, D), lambda i: (0, 0)),
            pl.BlockSpec((D, D // 2), lambda i: (0, 0)),
            pl.BlockSpec((D, D // 2), lambda i: (0, 0)),
        ],
        out_specs=[
            pl.BlockSpec((_RB, D), lambda i: (i, 0)),
            pl.BlockSpec((_RB, D // 2), lambda i: (i, 0)),
        ],
        out_shape=[
            jax.ShapeDtypeStruct((N, D), jnp.float32),
            jax.ShapeDtypeStruct((N, D // 2), jnp.int32),
        ],
    )(x, aggp, w, lnw, lnb, sel_e, sel_o)


def _tc_head_body(x_ref, wh_ref, b_ref, out_ref):
    i = pl.program_id(0)

    @pl.when(i == 0)
    def _():
        out_ref[...] = jnp.zeros_like(out_ref)

    e_node = jnp.dot(x_ref[...], wh_ref[...], preferred_element_type=jnp.float32)
    ids = lax.broadcasted_iota(jnp.int32, (_RB, NG), 1)
    onehot = (b_ref[...] == ids).astype(jnp.float32)
    out_ref[...] += lax.dot_general(
        onehot, e_node, (((0,), (0,)), ((), ())),
        preferred_element_type=jnp.float32)


def _tc_head(x, wh, batch2):
    return pl.pallas_call(
        _tc_head_body,
        grid=(_GN,),
        in_specs=[
            pl.BlockSpec((_RB, D), lambda i: (i, 0)),
            pl.BlockSpec((D, 1), lambda i: (0, 0)),
            pl.BlockSpec((_RB, 1), lambda i: (i, 0)),
        ],
        out_specs=pl.BlockSpec((NG, 1), lambda i: (0, 0)),
        out_shape=jax.ShapeDtypeStruct((NG, 1), jnp.float32),
    )(x, wh, batch2)


_QPERM = []
for _g in range(D // 32):
    _QPERM += list(range(32 * _g, 32 * _g + 32, 2))
    _QPERM += list(range(32 * _g + 1, 32 * _g + 32, 2))


def kernel(pos, node_atom, batch, edge_index, atom_table, rbf_centers,
           W_rbf1, W_rbf2, W_layers, ln_w, ln_b, W_head):
    src = edge_index[0].astype(jnp.int32)
    dst = edge_index[1].astype(jnp.int32)
    px = pos[:, 0]
    py = pos[:, 1]
    pz = pos[:, 2]
    zeros = jnp.zeros((RPT, D), jnp.float32)
    na2 = node_atom.astype(jnp.int32).reshape(N, 1)
    b2 = batch.astype(jnp.int32).reshape(N, 1)

    ar = jnp.arange(D // 2)
    sel_e = jnp.zeros((D, D // 2), jnp.float32).at[2 * ar, ar].set(1.0)
    sel_o = jnp.zeros((D, D // 2), jnp.float32).at[2 * ar + 1, ar].set(1.0)
    w_q = W_layers[:, jnp.array(_QPERM), :]

    d2 = _sc_d2(px, py, pz, src, dst)
    ewp = _tc_ew(d2.reshape(E, 1), rbf_centers.reshape(1, NB),
                 W_rbf1, W_rbf2, sel_e, sel_o)

    x, h = _tc_embed(na2, atom_table, ln_w[0].reshape(1, D),
                     ln_b[0].reshape(1, D), sel_e, sel_o)
    for l in range(L):
        aggp = _sc_agg(h, ewp, src, dst, zeros)
        nl = (l + 1) % L
        x, h = _tc_update(x, aggp, w_q[l], ln_w[nl].reshape(1, D),
                          ln_b[nl].reshape(1, D), sel_e, sel_o)
    return _tc_head(x, W_head, b2)


# speedup vs baseline: 1.5787x; 1.0974x over previous
import functools
import math

import jax
import jax.numpy as jnp
from jax import lax
from jax.experimental import pallas as pl
from jax.experimental.pallas import tpu as pltpu
import jax.experimental.pallas.tpu_sc as plsc

N = 10000
E = 320000
D = 128
NB = 128
L = 6
NG = 64
NATOM = 64
MAXR = 5.0
AVG_DEG = 32.0

NC = 2
NS = 16
NW = NC * NS
EPT = E // NW
C = 80
K = EPT // C
CA = 40
KA = EPT // CA
NP_ = 10240
RPT = NP_ // NS

_mesh = plsc.VectorSubcoreMesh(core_axis_name="c", subcore_axis_name="s")


@functools.partial(
    pl.kernel,
    out_type=jax.ShapeDtypeStruct((E,), jnp.float32),
    mesh=_mesh,
    compiler_params=pltpu.CompilerParams(needs_layout_passes=False),
    scratch_types=[
        pltpu.VMEM((EPT,), jnp.int32),
        pltpu.VMEM((EPT,), jnp.int32),
        pltpu.VMEM((6, C), jnp.float32),
        pltpu.VMEM((6, C), jnp.float32),
        pltpu.VMEM((EPT,), jnp.float32),
        pltpu.SemaphoreType.DMA,
        pltpu.SemaphoreType.DMA,
    ],
)
def _sc_d2(px_hbm, py_hbm, pz_hbm, src_hbm, dst_hbm, out_hbm,
           srcall, dstall, g0, g1, d2_v, s0, s1):
    c = lax.axis_index("c")
    s = lax.axis_index("s")
    w = c * NS + s
    base = w * EPT
    gb = (g0, g1)
    sb = (s0, s1)
    pltpu.sync_copy(src_hbm.at[pl.ds(base, EPT)], srcall)
    pltpu.sync_copy(dst_hbm.at[pl.ds(base, EPT)], dstall)

    def plan(k, b):
        si = srcall.at[pl.ds(k * C, C)]
        di = dstall.at[pl.ds(k * C, C)]
        return [
            (px_hbm.at[si], gb[b].at[0]),
            (py_hbm.at[si], gb[b].at[1]),
            (pz_hbm.at[si], gb[b].at[2]),
            (px_hbm.at[di], gb[b].at[3]),
            (py_hbm.at[di], gb[b].at[4]),
            (pz_hbm.at[di], gb[b].at[5]),
        ]

    def issue(k, b):
        for sr, ds_ in plan(k, b):
            pltpu.async_copy(sr, ds_, sb[b])

    def process(k, b):
        for sr, ds_ in plan(k, b):
            pltpu.make_async_copy(sr, ds_, sb[b]).wait()
        for t in range(C // 16):
            sl = pl.ds(t * 16, 16)
            dx = gb[b][0, sl] - gb[b][3, sl]
            dy = gb[b][1, sl] - gb[b][4, sl]
            dz = gb[b][2, sl] - gb[b][5, sl]
            d2_v[pl.ds(k * C + t * 16, 16)] = dx * dx + dy * dy + dz * dz

    issue(0, 0)
    issue(1, 1)

    def pair(i, _):
        process(2 * i, 0)
        issue(2 * i + 2, 0)
        process(2 * i + 1, 1)

        @pl.when(i < K // 2 - 1)
        def _():
            issue(2 * i + 3, 1)
        return 0

    lax.fori_loop(0, K // 2, pair, 0)
    process(K - 1, 0)
    pltpu.sync_copy(d2_v, out_hbm.at[pl.ds(base, EPT)])


_M16 = -65536


@functools.partial(
    pl.kernel,
    out_type=jax.ShapeDtypeStruct((NC, NP_, D), jnp.float32),
    mesh=_mesh,
    compiler_params=pltpu.CompilerParams(
        needs_layout_passes=False, use_tc_tiling_on_sc=False),
    scratch_types=[
        pltpu.VMEM_SHARED((NP_, D), jnp.float32),
        pltpu.VMEM((EPT,), jnp.int32),
        pltpu.VMEM((EPT,), jnp.int32),
        pltpu.VMEM((CA, D // 2), jnp.int32),
        pltpu.VMEM((CA, D // 2), jnp.int32),
        pltpu.VMEM((CA, D // 2), jnp.int32),
        pltpu.VMEM((CA, D // 2), jnp.int32),
        pltpu.VMEM((CA, D), jnp.float32),
        pltpu.VMEM((CA, D), jnp.float32),
        pltpu.SemaphoreType.DMA,
        pltpu.SemaphoreType.DMA,
        pltpu.SemaphoreType.DMA,
        pltpu.SemaphoreType.DMA,
        pltpu.SemaphoreType.DMA,
        pltpu.SemaphoreType.DMA,
    ],
)
def _sc_agg(h_hbm, ew_hbm, src_hbm, dst_hbm, zeros_hbm, out_hbm,
            agg_sh, srcall, dstall, msg0, msg1, ew0, ew1, sb0, sb1,
            sg0, sg1, se0, se1, ss0, ss1):
    c = lax.axis_index("c")
    s = lax.axis_index("s")
    w = c * NS + s
    base = w * EPT
    row0 = s * RPT
    msgb = (msg0, msg1)
    ewb = (ew0, ew1)
    sbb = (sb0, sb1)
    sgb = (sg0, sg1)
    seb = (se0, se1)
    ssb = (ss0, ss1)

    pltpu.sync_copy(zeros_hbm, agg_sh.at[pl.ds(row0, RPT)])
    pltpu.sync_copy(src_hbm.at[pl.ds(base, EPT)], srcall)
    pltpu.sync_copy(dst_hbm.at[pl.ds(base, EPT)], dstall)
    plsc.subcore_barrier()

    def issue(k, b):
        pltpu.async_copy(h_hbm.at[srcall.at[pl.ds(k * CA, CA)]], msgb[b], sgb[b])
        pltpu.async_copy(ew_hbm.at[pl.ds(base + k * CA, CA)], ewb[b], seb[b])

    def wait_scatter(b):
        pltpu.make_async_copy(
            sbb[b], agg_sh.at[dstall.at[pl.ds(0, CA)]], ssb[b]).wait()

    def process(k, b):
        pltpu.make_async_copy(
            h_hbm.at[srcall.at[pl.ds(0, CA)]], msgb[b], sgb[b]).wait()
        pltpu.make_async_copy(
            ew_hbm.at[pl.ds(0, CA)], ewb[b], seb[b]).wait()

        def rowbody(r, _):
            for j in range(D // 32):
                vh = msgb[b][r, pl.ds(j * 16, 16)]
                ve = ewb[b][pl.ds(r * (D // 2) + j * 16, 16)]
                he = plsc.bitcast(vh << 16, jnp.float32)
                ho = plsc.bitcast(vh & _M16, jnp.float32)
                ee = plsc.bitcast(ve << 16, jnp.float32)
                eo = plsc.bitcast(ve & _M16, jnp.float32)
                sbb[b][r, pl.ds(j * 32, 16)] = he * ee
                sbb[b][r, pl.ds(j * 32 + 16, 16)] = ho * eo
            return 0

        @plsc.parallel_loop(0, CA, 1, unroll=2)
        def rowbody(r):
            for j in range(D // 32):
                vh = msgb[b][r, pl.ds(j * 16, 16)]
                ve = ewb[b][r, pl.ds(j * 16, 16)]
                he = plsc.bitcast(vh << 16, jnp.float32)
                ho = plsc.bitcast(vh & _M16, jnp.float32)
                ee = plsc.bitcast(ve << 16, jnp.float32)
                eo = plsc.bitcast(ve & _M16, jnp.float32)
                sbb[b][r, pl.ds(j * 32, 16)] = he * ee
                sbb[b][r, pl.ds(j * 32 + 16, 16)] = ho * eo
        pltpu.async_copy(sbb[b], agg_sh.at[dstall.at[pl.ds(k * CA, CA)]],
                         ssb[b], add=True)

    issue(0, 0)
    issue(1, 1)

    def pair(i, _):
        for b in range(2):
            @pl.when(i > 0)
            def _():
                wait_scatter(b)

            process(2 * i + b, b)

            @pl.when(i + 1 < KA // 2)
            def _():
                issue(2 * i + 2 + b, b)
        return 0

    lax.fori_loop(0, KA // 2, pair, 0)
    wait_scatter(0)
    wait_scatter(1)
    plsc.subcore_barrier()
    pltpu.sync_copy(agg_sh.at[pl.ds(row0, RPT)],
                    out_hbm.at[c, pl.ds(row0, RPT)])


_RB = 2000
_GN = N // _RB
_EB = 2000
_GE = E // _EB


def _pack16(a, b):
    ai = lax.bitcast_convert_type(
        a.astype(jnp.bfloat16).astype(jnp.float32), jnp.int32)
    bi = lax.bitcast_convert_type(
        b.astype(jnp.bfloat16).astype(jnp.float32), jnp.int32)
    return lax.shift_right_logical(ai, 16) | (bi & _M16)


def _tc_ew_body(d2_ref, cen_ref, w1_ref, w2_ref, se_ref, so_ref, out_ref):
    d2 = d2_ref[...]
    dist = jnp.sqrt(d2 + 1e-8)
    cen = cen_ref[...]
    width = MAXR / NB
    rbf = jnp.exp(-((dist - cen) ** 2) * (1.0 / (2.0 * width * width)))
    h1 = jax.nn.silu(jnp.dot(rbf, w1_ref[...], preferred_element_type=jnp.float32))
    ew = jnp.dot(h1, w2_ref[...], preferred_element_type=jnp.float32)
    out_ref[...] = _pack16(
        jnp.dot(ew, se_ref[...], preferred_element_type=jnp.float32),
        jnp.dot(ew, so_ref[...], preferred_element_type=jnp.float32))


def _tc_ew(d2, centers, w1, w2, sel_e, sel_o):
    return pl.pallas_call(
        _tc_ew_body,
        grid=(_GE,),
        in_specs=[
            pl.BlockSpec((_EB, 1), lambda i: (i, 0)),
            pl.BlockSpec((1, NB), lambda i: (0, 0)),
            pl.BlockSpec((NB, 64), lambda i: (0, 0)),
            pl.BlockSpec((64, D), lambda i: (0, 0)),
            pl.BlockSpec((D, D // 2), lambda i: (0, 0)),
            pl.BlockSpec((D, D // 2), lambda i: (0, 0)),
        ],
        out_specs=pl.BlockSpec((_EB, D // 2), lambda i: (i, 0)),
        out_shape=jax.ShapeDtypeStruct((E, D // 2), jnp.int32),
    )(d2, centers, w1, w2, sel_e, sel_o)


def _layernorm(x, w, b):
    mu = jnp.mean(x, axis=-1, keepdims=True)
    var = jnp.var(x, axis=-1, keepdims=True)
    return (x - mu) / jnp.sqrt(var + 1e-5) * w + b


def _tc_embed_body(na_ref, tab_ref, lnw_ref, lnb_ref, se_ref, so_ref,
                   x_ref, h_ref):
    na = na_ref[...]
    ids = lax.broadcasted_iota(jnp.int32, (_RB, NATOM), 1)
    onehot = (na == ids).astype(jnp.float32)
    x = jnp.dot(onehot, tab_ref[...], preferred_element_type=jnp.float32)
    x_ref[...] = x
    h = _layernorm(x, lnw_ref[...], lnb_ref[...])
    h_ref[...] = _pack16(
        jnp.dot(h, se_ref[...], preferred_element_type=jnp.float32),
        jnp.dot(h, so_ref[...], preferred_element_type=jnp.float32))


def _tc_embed(na, tab, lnw0, lnb0, sel_e, sel_o):
    return pl.pallas_call(
        _tc_embed_body,
        grid=(_GN,),
        in_specs=[
            pl.BlockSpec((_RB, 1), lambda i: (i, 0)),
            pl.BlockSpec((NATOM, D), lambda i: (0, 0)),
            pl.BlockSpec((1, D), lambda i: (0, 0)),
            pl.BlockSpec((1, D), lambda i: (0, 0)),
            pl.BlockSpec((D, D // 2), lambda i: (0, 0)),
            pl.BlockSpec((D, D // 2), lambda i: (0, 0)),
        ],
        out_specs=[
            pl.BlockSpec((_RB, D), lambda i: (i, 0)),
            pl.BlockSpec((_RB, D // 2), lambda i: (i, 0)),
        ],
        out_shape=[
            jax.ShapeDtypeStruct((N, D), jnp.float32),
            jax.ShapeDtypeStruct((N, D // 2), jnp.int32),
        ],
    )(na, tab, lnw0, lnb0, sel_e, sel_o)


def _tc_update_body(x_ref, agg_ref, w_ref, lnw_ref, lnb_ref, se_ref, so_ref,
                    xn_ref, hn_ref):
    agg = (agg_ref[0] + agg_ref[1]) * (1.0 / math.sqrt(AVG_DEG))
    up = jnp.dot(agg, w_ref[...], preferred_element_type=jnp.float32)
    xn = x_ref[...] + jax.nn.silu(up)
    xn_ref[...] = xn
    hn = _layernorm(xn, lnw_ref[...], lnb_ref[...])
    hn_ref[...] = _pack16(
        jnp.dot(hn, se_ref[...], preferred_element_type=jnp.float32),
        jnp.dot(hn, so_ref[...], preferred_element_type=jnp.float32))


def _tc_update(x, aggp, w, lnw, lnb, sel_e, sel_o):
    return pl.pallas_call(
        _tc_update_body,
        grid=(_GN,),
        in_specs=[
            pl.BlockSpec((_RB, D), lambda i: (i, 0)),
            pl.BlockSpec((NC, _RB, D), lambda i: (0, i, 0)),
            pl.BlockSpec((D, D), lambda i: (0, 0)),
            pl.BlockSpec((1, D), lambda i: (0, 0)),
            pl.BlockSpec((1, D), lambda i: (0, 0)),
            pl.BlockSpec((D, D // 2), lambda i: (0, 0)),
            pl.BlockSpec((D, D // 2), lambda i: (0, 0)),
        ],
        out_specs=[
            pl.BlockSpec((_RB, D), lambda i: (i, 0)),
            pl.BlockSpec((_RB, D // 2), lambda i: (i, 0)),
        ],
        out_shape=[
            jax.ShapeDtypeStruct((N, D), jnp.float32),
            jax.ShapeDtypeStruct((N, D // 2), jnp.int32),
        ],
    )(x, aggp, w, lnw, lnb, sel_e, sel_o)


def _tc_head_body(x_ref, wh_ref, b_ref, out_ref):
    i = pl.program_id(0)

    @pl.when(i == 0)
    def _():
        out_ref[...] = jnp.zeros_like(out_ref)

    e_node = jnp.dot(x_ref[...], wh_ref[...], preferred_element_type=jnp.float32)
    ids = lax.broadcasted_iota(jnp.int32, (_RB, NG), 1)
    onehot = (b_ref[...] == ids).astype(jnp.float32)
    out_ref[...] += lax.dot_general(
        onehot, e_node, (((0,), (0,)), ((), ())),
        preferred_element_type=jnp.float32)


def _tc_head(x, wh, batch2):
    return pl.pallas_call(
        _tc_head_body,
        grid=(_GN,),
        in_specs=[
            pl.BlockSpec((_RB, D), lambda i: (i, 0)),
            pl.BlockSpec((D, 1), lambda i: (0, 0)),
            pl.BlockSpec((_RB, 1), lambda i: (i, 0)),
        ],
        out_specs=pl.BlockSpec((NG, 1), lambda i: (0, 0)),
        out_shape=jax.ShapeDtypeStruct((NG, 1), jnp.float32),
    )(x, wh, batch2)


_QPERM = []
for _g in range(D // 32):
    _QPERM += list(range(32 * _g, 32 * _g + 32, 2))
    _QPERM += list(range(32 * _g + 1, 32 * _g + 32, 2))


def kernel(pos, node_atom, batch, edge_index, atom_table, rbf_centers,
           W_rbf1, W_rbf2, W_layers, ln_w, ln_b, W_head):
    src = edge_index[0].astype(jnp.int32)
    dst = edge_index[1].astype(jnp.int32)
    px = pos[:, 0]
    py = pos[:, 1]
    pz = pos[:, 2]
    zeros = jnp.zeros((RPT, D), jnp.float32)
    na2 = node_atom.astype(jnp.int32).reshape(N, 1)
    b2 = batch.astype(jnp.int32).reshape(N, 1)

    ar = jnp.arange(D // 2)
    sel_e = jnp.zeros((D, D // 2), jnp.float32).at[2 * ar, ar].set(1.0)
    sel_o = jnp.zeros((D, D // 2), jnp.float32).at[2 * ar + 1, ar].set(1.0)
    w_q = W_layers[:, jnp.array(_QPERM), :]

    d2 = _sc_d2(px, py, pz, src, dst)
    ewp = _tc_ew(d2.reshape(E, 1), rbf_centers.reshape(1, NB),
                 W_rbf1, W_rbf2, sel_e, sel_o)

    x, h = _tc_embed(na2, atom_table, ln_w[0].reshape(1, D),
                     ln_b[0].reshape(1, D), sel_e, sel_o)
    for l in range(L):
        aggp = _sc_agg(h, ewp, src, dst, zeros)
        nl = (l + 1) % L
        x, h = _tc_update(x, aggp, w_q[l], ln_w[nl].reshape(1, D),
                          ln_b[nl].reshape(1, D), sel_e, sel_o)
    return _tc_head(x, W_head, b2)
